# Initial kernel scaffold; baseline (speedup 1.0000x reference)
import jax
import jax.numpy as jnp
from jax.experimental import pallas as pl


def kernel(triples, ent_emb, rel_emb, neigh_table, neigh_lens):
    raise NotImplementedError("write your pallas kernel here")


# speedup vs baseline: 1.5175x; 1.5175x over previous
import functools

import jax
import jax.numpy as jnp
from jax import lax
from jax.experimental import pallas as pl
from jax.experimental.pallas import tpu as pltpu
from jax.experimental.pallas import tpu_sc as plsc

NC = 2
NS = 16
NW = NC * NS
LANES = 16

DIM = 64
NEI = 9
NEI_PAD = 16
G = 16


def _sc_gather_pool(ids, r_ids, neigh16, ent_emb, rel_emb):
    two_b = ids.shape[0]
    b = two_b // 2
    s_half = b // NW
    ng = s_half // G
    rel_per_w = b // NW

    mesh = plsc.VectorSubcoreMesh(core_axis_name="c", subcore_axis_name="s")

    @functools.partial(
        pl.kernel,
        out_type=(
            jax.ShapeDtypeStruct((b, DIM), jnp.float32),
            jax.ShapeDtypeStruct((b, DIM), jnp.float32),
            jax.ShapeDtypeStruct((b, DIM), jnp.float32),
        ),
        mesh=mesh,
        scratch_types=[
            pltpu.VMEM((2 * s_half,), jnp.int32),
            pltpu.VMEM((G, NEI_PAD), jnp.int32),
            pltpu.VMEM((NEI, G, DIM), jnp.float32),
            pltpu.VMEM((G, DIM), jnp.float32),
            pltpu.VMEM((rel_per_w,), jnp.int32),
            pltpu.VMEM((rel_per_w, DIM), jnp.float32),
            pltpu.SemaphoreType.DMA,
            pltpu.SemaphoreType.DMA,
        ],
        compiler_params=pltpu.CompilerParams(use_tc_tiling_on_sc=False,
                                             needs_layout_passes=False),
    )
    def k(ids_hbm, rid_hbm, neigh_hbm, ent_hbm, rel_hbm,
          hsum_out, tsum_out, rrow_out,
          ids_v, neigh_v, emb_v, acc_v, rid_v, rrow_v, sem_n, sem_e):
        wid = lax.axis_index("s") * NC + lax.axis_index("c")

        pltpu.sync_copy(ids_hbm.at[pl.ds(wid * s_half, s_half)],
                        ids_v.at[pl.ds(0, s_half)])
        pltpu.sync_copy(ids_hbm.at[pl.ds(b + wid * s_half, s_half)],
                        ids_v.at[pl.ds(s_half, s_half)])

        lane = lax.iota(jnp.int32, LANES)

        def do_side(id_off, out_hbm):
            out_base = wid * s_half

            def group(gi, carry):
                gvec = ids_v[pl.ds(id_off + gi * G, G)]
                pltpu.async_copy(neigh_hbm.at[gvec], neigh_v, sem_n).wait()
                descs = []
                for j in range(NEI):
                    col = plsc.load_gather(
                        neigh_v, [lane, jnp.full((LANES,), j, jnp.int32)])
                    descs.append(pltpu.async_copy(
                        ent_hbm.at[col], emb_v.at[j], sem_e))
                for d in descs:
                    d.wait()
                for g in range(G):
                    for c in range(DIM // LANES):
                        sl = pl.ds(c * LANES, LANES)
                        acc = emb_v[0, g, sl]
                        for j in range(1, NEI):
                            acc = acc + emb_v[j, g, sl]
                        acc_v[g, sl] = acc
                pltpu.sync_copy(
                    acc_v, out_hbm.at[pl.ds(out_base + gi * G, G)])
                return carry

            lax.fori_loop(0, ng, group, 0)

        do_side(0, hsum_out)
        do_side(s_half, tsum_out)

        pltpu.sync_copy(rid_hbm.at[pl.ds(wid * rel_per_w, rel_per_w)], rid_v)
        pltpu.async_copy(rel_hbm.at[rid_v], rrow_v, sem_e).wait()
        pltpu.sync_copy(rrow_v, rrow_out.at[pl.ds(wid * rel_per_w, rel_per_w)])

    return k(ids, r_ids, neigh16, ent_emb, rel_emb)


def _tc_score(hsum, rrow, tsum):
    b = hsum.shape[0]
    blk = 2048

    def body(h_ref, r_ref, t_ref, o_ref):
        def nrm(x):
            n2 = jnp.sum(x * x, axis=1, keepdims=True)
            return x / jnp.maximum(jnp.sqrt(n2), 1e-12)

        v = nrm(h_ref[...]) + nrm(r_ref[...]) - nrm(t_ref[...])
        o_ref[...] = jnp.sum(jnp.abs(v), axis=1)

    return pl.pallas_call(
        body,
        grid=(b // blk,),
        in_specs=[
            pl.BlockSpec((blk, DIM), lambda i: (i, 0)),
            pl.BlockSpec((blk, DIM), lambda i: (i, 0)),
            pl.BlockSpec((blk, DIM), lambda i: (i, 0)),
        ],
        out_specs=pl.BlockSpec((blk,), lambda i: (i,)),
        out_shape=jax.ShapeDtypeStruct((b,), jnp.float32),
    )(hsum, rrow, tsum)


def kernel(triples, ent_emb, rel_emb, neigh_table, neigh_lens):
    del neigh_lens
    h_ids = triples[:, 0]
    r_ids = triples[:, 1]
    t_ids = triples[:, 2]
    ids = jnp.concatenate([h_ids, t_ids], axis=0)
    neigh16 = jnp.pad(neigh_table, ((0, 0), (0, NEI_PAD - NEI)))
    hsum, tsum, rrow = _sc_gather_pool(ids, r_ids, neigh16, ent_emb, rel_emb)
    return _tc_score(hsum, rrow, tsum)
